# baseline (device time: 48073 ns/iter reference)
import jax
import jax.numpy as jnp
from jax import lax
from jax.experimental import pallas as pl
from jax.experimental.pallas import tpu as pltpu

N_DEV = 8
B = 2
SQ = 512
DMODEL = 768
DH = 64
H_LOC = 8
QCOLS = H_LOC * DH
CH = SQ // N_DEV
F32 = jnp.float32
BF16 = jnp.bfloat16


def kernel(x, Wq, K_ext, V_ext, Wo):
    my_i = lax.axis_index("i")
    wq_sl = lax.dynamic_slice(Wq, (0, my_i * QCOLS), (DMODEL, QCOLS))
    wo_sl = lax.dynamic_slice(Wo, (my_i * QCOLS, 0), (QCOLS, DMODEL))

    def body(x_ref, wq_ref, k_ref, v_ref, wo_ref, out_ref,
             wq_bf, wo_bf, q_scr, k_scr, v_scr, send_ref, a2a_ref, acc_ref,
             rows_ref, ag_ref,
             a2a_send_sems, a2a_recv_sems, ag_send_sems, ag_recv_sems):
        my = lax.axis_index("i")

        barrier_sem = pltpu.get_barrier_semaphore()
        for r in range(1, N_DEV):
            pl.semaphore_signal(
                barrier_sem, inc=1,
                device_id=(lax.rem(my + r, N_DEV),),
                device_id_type=pl.DeviceIdType.MESH,
            )
        pl.semaphore_wait(barrier_sem, N_DEV - 1)

        wq_bf[...] = wq_ref[...].astype(BF16)
        wo_bf[...] = wo_ref[...].astype(BF16)
        acc_ref[...] = jnp.zeros((B, CH, DMODEL), F32)

        for b in range(B):
            qf = jnp.dot(x_ref[b].astype(BF16), wq_bf[...],
                         preferred_element_type=F32)
            q_scr[b] = jnp.transpose(
                (qf * 0.125).astype(BF16).reshape(SQ, H_LOC, DH), (1, 0, 2))
            k_scr[b] = jnp.transpose(k_ref[b].astype(BF16), (1, 0, 2))
            v_scr[b] = jnp.transpose(v_ref[b].astype(BF16), (1, 0, 2))

        a2a_rdmas = {}

        def a2a_send(r):
            rdma = pltpu.make_async_remote_copy(
                src_ref=send_ref.at[r - 1],
                dst_ref=a2a_ref.at[r - 1],
                send_sem=a2a_send_sems.at[r - 1],
                recv_sem=a2a_recv_sems.at[r - 1],
                device_id=(lax.rem(my + r, N_DEV),),
                device_id_type=pl.DeviceIdType.MESH,
            )
            rdma.start()
            a2a_rdmas[r] = rdma

        def add_partial(r):
            a2a_rdmas[r].wait_recv()
            for b in range(B):
                acc_ref[b] = acc_ref[b] + a2a_ref[r - 1, b].astype(F32)

        for d in range(1, 5):
            c_a = lax.rem(my + d, N_DEV)
            c_b = lax.rem(my + d + 4, N_DEV)
            g4 = lax.rem(c_a, 4)
            for b in range(B):
                q3 = jnp.concatenate(
                    [q_scr[b, :, pl.ds(c_a * 64, 64), :],
                     q_scr[b, :, pl.ds(c_b * 64, 64), :]], axis=1)
                k3 = jnp.concatenate(
                    [k_scr[b, :, pl.ds(g4 * 64, 64), :],
                     k_scr[b, :, pl.ds(g4 * 64 + 256, 64), :]], axis=1)
                v3 = jnp.concatenate(
                    [v_scr[b, :, pl.ds(g4 * 64, 64), :],
                     v_scr[b, :, pl.ds(g4 * 64 + 256, 64), :]], axis=1)
                s = lax.dot_general(
                    q3, k3, (((2,), (2,)), ((0,), (0,))),
                    preferred_element_type=F32)
                w = jnp.exp(s)
                denom = jnp.sum(w, axis=-1, keepdims=True)
                ctx3 = lax.dot_general(
                    w.astype(BF16), v3, (((2,), (1,)), ((0,), (0,))),
                    preferred_element_type=F32)
                ctx3 = ctx3 / denom
                ctx = jnp.transpose(ctx3, (1, 0, 2)).reshape(128, QCOLS)
                part = jnp.dot(ctx.astype(BF16), wo_bf[...],
                               preferred_element_type=F32)
                send_ref[d - 1, b] = part[:64].astype(BF16)
                if d < 4:
                    send_ref[d + 3, b] = part[64:].astype(BF16)
                else:
                    acc_ref[b] = acc_ref[b] + part[64:]
            a2a_send(d)
            if d < 4:
                a2a_send(d + 4)
            if d >= 2:
                add_partial(d - 1)
                add_partial(d + 3)

        add_partial(4)

        out_ref[:, pl.ds(my * CH, CH), :] = acc_ref[...]
        rows_ref[...] = acc_ref[...].astype(BF16)
        ag_rdmas = {}
        for r in range(1, N_DEV):
            rdma = pltpu.make_async_remote_copy(
                src_ref=rows_ref,
                dst_ref=ag_ref.at[r - 1],
                send_sem=ag_send_sems.at[r - 1],
                recv_sem=ag_recv_sems.at[r - 1],
                device_id=(lax.rem(my + r, N_DEV),),
                device_id_type=pl.DeviceIdType.MESH,
            )
            rdma.start()
            ag_rdmas[r] = rdma
        for r in range(1, N_DEV):
            ag_rdmas[r].wait_recv()
            src_dev = lax.rem(my - r + N_DEV, N_DEV)
            out_ref[:, pl.ds(src_dev * CH, CH), :] = ag_ref[r - 1].astype(F32)

        for r in range(1, N_DEV):
            a2a_rdmas[r].wait_send()
            ag_rdmas[r].wait_send()

    return pl.pallas_call(
        body,
        out_shape=jax.ShapeDtypeStruct((B, SQ, DMODEL), F32),
        in_specs=[pl.BlockSpec(memory_space=pltpu.VMEM)] * 5,
        out_specs=pl.BlockSpec(memory_space=pltpu.VMEM),
        scratch_shapes=[
            pltpu.VMEM((DMODEL, QCOLS), BF16),
            pltpu.VMEM((QCOLS, DMODEL), BF16),
            pltpu.VMEM((B, H_LOC, SQ, DH), BF16),
            pltpu.VMEM((B, H_LOC, SQ, DH), BF16),
            pltpu.VMEM((B, H_LOC, SQ, DH), BF16),
            pltpu.VMEM((N_DEV - 1, B, CH, DMODEL), BF16),
            pltpu.VMEM((N_DEV - 1, B, CH, DMODEL), BF16),
            pltpu.VMEM((B, CH, DMODEL), F32),
            pltpu.VMEM((B, CH, DMODEL), BF16),
            pltpu.VMEM((N_DEV - 1, B, CH, DMODEL), BF16),
            pltpu.SemaphoreType.DMA((N_DEV - 1,)),
            pltpu.SemaphoreType.DMA((N_DEV - 1,)),
            pltpu.SemaphoreType.DMA((N_DEV - 1,)),
            pltpu.SemaphoreType.DMA((N_DEV - 1,)),
        ],
        compiler_params=pltpu.CompilerParams(collective_id=0),
    )(x, wq_sl, K_ext, V_ext, wo_sl)


# device time: 42265 ns/iter; 1.1374x vs baseline; 1.1374x over previous
import jax
import jax.numpy as jnp
from jax import lax
from jax.experimental import pallas as pl
from jax.experimental.pallas import tpu as pltpu

N_DEV = 8
B = 2
SQ = 512
DMODEL = 768
DH = 64
H_LOC = 8
QCOLS = H_LOC * DH
CH = SQ // N_DEV
F32 = jnp.float32
BF16 = jnp.bfloat16


def kernel(x, Wq, K_ext, V_ext, Wo):
    my_i = lax.axis_index("i")
    wq_sl = lax.dynamic_slice(Wq, (0, my_i * QCOLS), (DMODEL, QCOLS))
    wo_sl = lax.dynamic_slice(Wo, (my_i * QCOLS, 0), (QCOLS, DMODEL))

    def body(x_ref, wq_ref, k_ref, v_ref, wo_ref, out_ref,
             wq_bf, wo_bf, q_scr, k_scr, v_scr, send_ref, a2a_ref, acc_ref,
             rows_ref, ag_ref, a2a_send_sems, a2a_recv_sems, ag_send_sems, ag_recv_sems):
        my = lax.axis_index("i")

        barrier_sem = pltpu.get_barrier_semaphore()
        for r in range(1, N_DEV):
            pl.semaphore_signal(
                barrier_sem, inc=1,
                device_id=(lax.rem(my + r, N_DEV),),
                device_id_type=pl.DeviceIdType.MESH,
            )
        pl.semaphore_wait(barrier_sem, N_DEV - 1)

        wq_bf[...] = wq_ref[...].astype(BF16)
        wo_bf[...] = wo_ref[...].astype(BF16)

        for b in range(B):
            qf = jnp.dot(x_ref[b].astype(BF16), wq_bf[...],
                         preferred_element_type=F32)
            q_scr[b] = jnp.transpose(
                (qf * 0.125).astype(BF16).reshape(SQ, H_LOC, DH), (1, 0, 2))
            k_scr[b] = jnp.transpose(k_ref[b].astype(BF16), (1, 0, 2))
            v_scr[b] = jnp.transpose(v_ref[b].astype(BF16), (1, 0, 2))

        a2a_rdmas = {}

        def a2a_send(r):
            rdma = pltpu.make_async_remote_copy(
                src_ref=send_ref.at[r - 1],
                dst_ref=a2a_ref.at[r - 1],
                send_sem=a2a_send_sems.at[r - 1],
                recv_sem=a2a_recv_sems.at[r - 1],
                device_id=(lax.rem(my + r, N_DEV),),
                device_id_type=pl.DeviceIdType.MESH,
            )
            rdma.start()
            a2a_rdmas[r] = rdma

        for d in range(1, 5):
            c_a = lax.rem(my + d, N_DEV)
            c_b = lax.rem(my + d + 4, N_DEV)
            g4 = lax.rem(c_a, 4)
            for b in range(B):
                q3 = jnp.concatenate(
                    [q_scr[b, :, pl.ds(c_a * 64, 64), :],
                     q_scr[b, :, pl.ds(c_b * 64, 64), :]], axis=1)
                k3 = jnp.concatenate(
                    [k_scr[b, :, pl.ds(g4 * 64, 64), :],
                     k_scr[b, :, pl.ds(g4 * 64 + 256, 64), :]], axis=1)
                v3 = jnp.concatenate(
                    [v_scr[b, :, pl.ds(g4 * 64, 64), :],
                     v_scr[b, :, pl.ds(g4 * 64 + 256, 64), :]], axis=1)
                s = lax.dot_general(
                    q3, k3, (((2,), (2,)), ((0,), (0,))),
                    preferred_element_type=F32)
                w = jnp.exp(s)
                denom = jnp.sum(w, axis=-1, keepdims=True)
                ctx3 = lax.dot_general(
                    w.astype(BF16), v3, (((2,), (1,)), ((0,), (0,))),
                    preferred_element_type=F32)
                ctx3 = ctx3 / denom
                ctx = jnp.transpose(ctx3, (1, 0, 2)).reshape(128, QCOLS)
                part = jnp.dot(ctx.astype(BF16), wo_bf[...],
                               preferred_element_type=F32)
                send_ref[d - 1, b] = part[:64].astype(BF16)
                if d < 4:
                    send_ref[d + 3, b] = part[64:].astype(BF16)
                else:
                    acc_ref[b] = part[64:]
            a2a_send(d)
            if d < 4:
                a2a_send(d + 4)

        for r in (1, 5, 2, 6, 3, 7, 4):
            a2a_rdmas[r].wait_recv()
            for b in range(B):
                acc_ref[b] = acc_ref[b] + a2a_ref[r - 1, b].astype(F32)

        out_ref[:, pl.ds(my * CH, CH), :] = acc_ref[...]
        rows_ref[...] = acc_ref[...].astype(BF16)
        ag_rdmas = {}
        for r in range(1, N_DEV):
            rdma = pltpu.make_async_remote_copy(
                src_ref=rows_ref,
                dst_ref=ag_ref.at[r - 1],
                send_sem=ag_send_sems.at[r - 1],
                recv_sem=ag_recv_sems.at[r - 1],
                device_id=(lax.rem(my + r, N_DEV),),
                device_id_type=pl.DeviceIdType.MESH,
            )
            rdma.start()
            ag_rdmas[r] = rdma
        for r in range(1, N_DEV):
            ag_rdmas[r].wait_recv()
            src_dev = lax.rem(my - r + N_DEV, N_DEV)
            out_ref[:, pl.ds(src_dev * CH, CH), :] = ag_ref[r - 1].astype(F32)

        for r in range(1, N_DEV):
            a2a_rdmas[r].wait_send()
            ag_rdmas[r].wait_send()

    return pl.pallas_call(
        body,
        out_shape=jax.ShapeDtypeStruct((B, SQ, DMODEL), F32),
        in_specs=[pl.BlockSpec(memory_space=pltpu.VMEM)] * 5,
        out_specs=pl.BlockSpec(memory_space=pltpu.VMEM),
        scratch_shapes=[
            pltpu.VMEM((DMODEL, QCOLS), BF16),
            pltpu.VMEM((QCOLS, DMODEL), BF16),
            pltpu.VMEM((B, H_LOC, SQ, DH), BF16),
            pltpu.VMEM((B, H_LOC, SQ, DH), BF16),
            pltpu.VMEM((B, H_LOC, SQ, DH), BF16),
            pltpu.VMEM((N_DEV - 1, B, CH, DMODEL), BF16),
            pltpu.VMEM((N_DEV - 1, B, CH, DMODEL), BF16),
            pltpu.VMEM((B, CH, DMODEL), F32),
            pltpu.VMEM((B, CH, DMODEL), BF16),
            pltpu.VMEM((N_DEV - 1, B, CH, DMODEL), BF16),
            pltpu.SemaphoreType.DMA((N_DEV - 1,)),
            pltpu.SemaphoreType.DMA((N_DEV - 1,)),
            pltpu.SemaphoreType.DMA((N_DEV - 1,)),
            pltpu.SemaphoreType.DMA((N_DEV - 1,)),
        ],
        compiler_params=pltpu.CompilerParams(collective_id=0),
    )(x, wq_sl, K_ext, V_ext, wo_sl)


# device time: 39121 ns/iter; 1.2288x vs baseline; 1.0804x over previous
import jax
import jax.numpy as jnp
from jax import lax
from jax.experimental import pallas as pl
from jax.experimental.pallas import tpu as pltpu

N_DEV = 8
B = 2
SQ = 512
DMODEL = 768
DH = 64
H_LOC = 8
QCOLS = H_LOC * DH
CH = SQ // N_DEV
F32 = jnp.float32
BF16 = jnp.bfloat16


def kernel(x, Wq, K_ext, V_ext, Wo):
    my_i = lax.axis_index("i")
    wq_sl = lax.dynamic_slice(Wq, (0, my_i * QCOLS), (DMODEL, QCOLS))
    wo_sl = lax.dynamic_slice(Wo, (my_i * QCOLS, 0), (QCOLS, DMODEL))

    def body(x_ref, wq_ref, k_ref, v_ref, wo_ref, out_ref,
             wq_bf, wo_bf, q_scr, send_ref, a2a_ref, acc_ref, rows_ref,
             ag_ref, a2a_send_sems, a2a_recv_sems, ag_send_sems, ag_recv_sems):
        my = lax.axis_index("i")

        barrier_sem = pltpu.get_barrier_semaphore()
        for r in range(1, N_DEV):
            pl.semaphore_signal(
                barrier_sem, inc=1,
                device_id=(lax.rem(my + r, N_DEV),),
                device_id_type=pl.DeviceIdType.MESH,
            )

        wq_bf[...] = wq_ref[...].astype(BF16)
        wo_bf[...] = wo_ref[...].astype(BF16)

        for b in range(B):
            qf = jnp.dot(x_ref[b].astype(BF16), wq_bf[...],
                         preferred_element_type=F32)
            q_scr[b] = (qf * 0.125).astype(BF16)

        pl.semaphore_wait(barrier_sem, N_DEV - 1)

        a2a_rdmas = {}

        def a2a_send(r):
            rdma = pltpu.make_async_remote_copy(
                src_ref=send_ref.at[r - 1],
                dst_ref=a2a_ref.at[r - 1],
                send_sem=a2a_send_sems.at[r - 1],
                recv_sem=a2a_recv_sems.at[r - 1],
                device_id=(lax.rem(my + r, N_DEV),),
                device_id_type=pl.DeviceIdType.MESH,
            )
            rdma.start()
            a2a_rdmas[r] = rdma

        for d in range(1, 5):
            c_a = lax.rem(my + d, N_DEV)
            c_b = lax.rem(my + d + 4, N_DEV)
            g4 = lax.rem(c_a, 4)
            for b in range(B):
                qa = q_scr[b, pl.ds(c_a * 64, 64), :]
                qb = q_scr[b, pl.ds(c_b * 64, 64), :]
                qg = jnp.concatenate([qa, qb], axis=0)
                q3 = jnp.transpose(qg.reshape(128, H_LOC, DH),
                                   (1, 0, 2))
                k1 = k_ref[b, pl.ds(g4 * 64, 64), :, :]
                k2 = k_ref[b, pl.ds(g4 * 64 + 256, 64), :, :]
                v1 = v_ref[b, pl.ds(g4 * 64, 64), :, :]
                v2 = v_ref[b, pl.ds(g4 * 64 + 256, 64), :, :]
                k3 = jnp.transpose(
                    jnp.concatenate([k1, k2], axis=0).astype(BF16),
                    (1, 0, 2))
                v3 = jnp.transpose(
                    jnp.concatenate([v1, v2], axis=0).astype(BF16),
                    (1, 0, 2))
                s = lax.dot_general(
                    q3, k3, (((2,), (2,)), ((0,), (0,))),
                    preferred_element_type=F32)
                w = jnp.exp(s)
                denom = jnp.sum(w, axis=-1, keepdims=True)
                ctx3 = lax.dot_general(
                    w.astype(BF16), v3, (((2,), (1,)), ((0,), (0,))),
                    preferred_element_type=F32)
                ctx3 = ctx3 / denom
                ctx = jnp.transpose(ctx3, (1, 0, 2)).reshape(128, QCOLS)
                part = jnp.dot(ctx.astype(BF16), wo_bf[...],
                               preferred_element_type=F32)
                send_ref[d - 1, b] = part[:64].astype(BF16)
                if d < 4:
                    send_ref[d + 3, b] = part[64:].astype(BF16)
                else:
                    acc_ref[b] = part[64:]
            a2a_send(d)
            if d < 4:
                a2a_send(d + 4)

        for r in (1, 5, 2, 6, 3, 7, 4):
            a2a_rdmas[r].wait_recv()
            for b in range(B):
                acc_ref[b] = acc_ref[b] + a2a_ref[r - 1, b].astype(F32)

        out_ref[:, pl.ds(my * CH, CH), :] = acc_ref[...]
        rows_ref[...] = acc_ref[...].astype(BF16)
        ag_rdmas = {}
        for r in range(1, N_DEV):
            rdma = pltpu.make_async_remote_copy(
                src_ref=rows_ref,
                dst_ref=ag_ref.at[r - 1],
                send_sem=ag_send_sems.at[r - 1],
                recv_sem=ag_recv_sems.at[r - 1],
                device_id=(lax.rem(my + r, N_DEV),),
                device_id_type=pl.DeviceIdType.MESH,
            )
            rdma.start()
            ag_rdmas[r] = rdma
        for r in range(1, N_DEV):
            ag_rdmas[r].wait_recv()
            src_dev = lax.rem(my - r + N_DEV, N_DEV)
            out_ref[:, pl.ds(src_dev * CH, CH), :] = ag_ref[r - 1].astype(F32)

        for r in range(1, N_DEV):
            a2a_rdmas[r].wait_send()
            ag_rdmas[r].wait_send()

    return pl.pallas_call(
        body,
        out_shape=jax.ShapeDtypeStruct((B, SQ, DMODEL), F32),
        in_specs=[pl.BlockSpec(memory_space=pltpu.VMEM)] * 5,
        out_specs=pl.BlockSpec(memory_space=pltpu.VMEM),
        scratch_shapes=[
            pltpu.VMEM((DMODEL, QCOLS), BF16),
            pltpu.VMEM((QCOLS, DMODEL), BF16),
            pltpu.VMEM((B, SQ, QCOLS), BF16),
            pltpu.VMEM((N_DEV - 1, B, CH, DMODEL), BF16),
            pltpu.VMEM((N_DEV - 1, B, CH, DMODEL), BF16),
            pltpu.VMEM((B, CH, DMODEL), F32),
            pltpu.VMEM((B, CH, DMODEL), BF16),
            pltpu.VMEM((N_DEV - 1, B, CH, DMODEL), BF16),
            pltpu.SemaphoreType.DMA((N_DEV - 1,)),
            pltpu.SemaphoreType.DMA((N_DEV - 1,)),
            pltpu.SemaphoreType.DMA((N_DEV - 1,)),
            pltpu.SemaphoreType.DMA((N_DEV - 1,)),
        ],
        compiler_params=pltpu.CompilerParams(collective_id=0),
    )(x, wq_sl, K_ext, V_ext, wo_sl)


# device time: 38955 ns/iter; 1.2341x vs baseline; 1.0043x over previous
import jax
import jax.numpy as jnp
from jax import lax
from jax.experimental import pallas as pl
from jax.experimental.pallas import tpu as pltpu

N_DEV = 8
B = 2
SQ = 512
DMODEL = 768
DH = 64
H_LOC = 8
QCOLS = H_LOC * DH
CH = SQ // N_DEV
F32 = jnp.float32
BF16 = jnp.bfloat16


def kernel(x, Wq, K_ext, V_ext, Wo):
    my_i = lax.axis_index("i")
    wq_sl = lax.dynamic_slice(Wq, (0, my_i * QCOLS), (DMODEL, QCOLS))
    wo_sl = lax.dynamic_slice(Wo, (my_i * QCOLS, 0), (QCOLS, DMODEL))

    def body(x_ref, wq_ref, k_ref, v_ref, wo_ref, out_ref,
             wq_bf, wo_bf, q_scr, send_ref, a2a_ref, acc_ref, rows_ref,
             ag_ref, a2a_send_sems, a2a_recv_sems, ag_send_sems, ag_recv_sems):
        my = lax.axis_index("i")

        barrier_sem = pltpu.get_barrier_semaphore()
        for r in range(1, N_DEV):
            pl.semaphore_signal(
                barrier_sem, inc=1,
                device_id=(lax.rem(my + r, N_DEV),),
                device_id_type=pl.DeviceIdType.MESH,
            )

        wq_bf[...] = wq_ref[...].astype(BF16)
        wo_bf[...] = wo_ref[...].astype(BF16)

        for b in range(B):
            qf = jnp.dot(x_ref[b].astype(BF16), wq_bf[...],
                         preferred_element_type=F32)
            q_scr[b] = (qf * 0.125).astype(BF16)

        pl.semaphore_wait(barrier_sem, N_DEV - 1)

        a2a_rdmas = {}

        def a2a_send(r):
            rdma = pltpu.make_async_remote_copy(
                src_ref=send_ref.at[r - 1],
                dst_ref=a2a_ref.at[r - 1],
                send_sem=a2a_send_sems.at[r - 1],
                recv_sem=a2a_recv_sems.at[r - 1],
                device_id=(lax.rem(my + r, N_DEV),),
                device_id_type=pl.DeviceIdType.MESH,
            )
            rdma.start()
            a2a_rdmas[r] = rdma

        def kv3_for(b, g4):
            k1 = k_ref[b, pl.ds(g4 * 64, 64), :, :]
            k2 = k_ref[b, pl.ds(g4 * 64 + 256, 64), :, :]
            v1 = v_ref[b, pl.ds(g4 * 64, 64), :, :]
            v2 = v_ref[b, pl.ds(g4 * 64 + 256, 64), :, :]
            k3 = jnp.transpose(
                jnp.concatenate([k1, k2], axis=0).astype(BF16),
                (1, 0, 2))
            v3 = jnp.transpose(
                jnp.concatenate([v1, v2], axis=0).astype(BF16),
                (1, 0, 2))
            return k3, v3

        def attend(q3, k3, v3):
            s = lax.dot_general(
                q3, k3, (((2,), (2,)), ((0,), (0,))),
                preferred_element_type=F32)
            w = jnp.exp(s)
            denom = jnp.sum(w, axis=-1, keepdims=True)
            ctx3 = lax.dot_general(
                w.astype(BF16), v3, (((2,), (1,)), ((0,), (0,))),
                preferred_element_type=F32)
            ctx3 = ctx3 / denom
            rows = ctx3.shape[1]
            ctx = jnp.transpose(ctx3, (1, 0, 2)).reshape(rows, QCOLS)
            return jnp.dot(ctx.astype(BF16), wo_bf[...],
                           preferred_element_type=F32)

        def q3_rows(b, c, n):
            qr = q_scr[b, pl.ds(c * 64, 64 * n), :]
            return jnp.transpose(qr.reshape(64 * n, H_LOC, DH), (1, 0, 2))

        for d in range(1, 4):
            c_a = lax.rem(my + d, N_DEV)
            c_b = lax.rem(my + d + 4, N_DEV)
            g4 = lax.rem(c_a, 4)
            for b in range(B):
                qg = jnp.concatenate(
                    [q_scr[b, pl.ds(c_a * 64, 64), :],
                     q_scr[b, pl.ds(c_b * 64, 64), :]], axis=0)
                q3 = jnp.transpose(qg.reshape(128, H_LOC, DH), (1, 0, 2))
                k3, v3 = kv3_for(b, g4)
                part = attend(q3, k3, v3)
                send_ref[d - 1, b] = part[:64].astype(BF16)
                send_ref[d + 3, b] = part[64:].astype(BF16)
            a2a_send(d)
            a2a_send(d + 4)

        c_a = lax.rem(my + 4, N_DEV)
        c_b = my
        g4 = lax.rem(c_a, 4)
        kv = [kv3_for(b, g4) for b in range(B)]
        for b in range(B):
            part = attend(q3_rows(b, c_a, 1), *kv[b])
            send_ref[3, b] = part.astype(BF16)
        a2a_send(4)
        for b in range(B):
            acc_ref[b] = attend(q3_rows(b, c_b, 1), *kv[b])

        for r in (1, 5, 2, 6, 3, 7, 4):
            a2a_rdmas[r].wait_recv()
            for b in range(B):
                acc_ref[b] = acc_ref[b] + a2a_ref[r - 1, b].astype(F32)

        out_ref[:, pl.ds(my * CH, CH), :] = acc_ref[...]
        rows_ref[...] = acc_ref[...].astype(BF16)
        ag_rdmas = {}
        for r in range(1, N_DEV):
            rdma = pltpu.make_async_remote_copy(
                src_ref=rows_ref,
                dst_ref=ag_ref.at[r - 1],
                send_sem=ag_send_sems.at[r - 1],
                recv_sem=ag_recv_sems.at[r - 1],
                device_id=(lax.rem(my + r, N_DEV),),
                device_id_type=pl.DeviceIdType.MESH,
            )
            rdma.start()
            ag_rdmas[r] = rdma
        for r in range(1, N_DEV):
            ag_rdmas[r].wait_recv()
            src_dev = lax.rem(my - r + N_DEV, N_DEV)
            out_ref[:, pl.ds(src_dev * CH, CH), :] = ag_ref[r - 1].astype(F32)

        for r in range(1, N_DEV):
            a2a_rdmas[r].wait_send()
            ag_rdmas[r].wait_send()

    return pl.pallas_call(
        body,
        out_shape=jax.ShapeDtypeStruct((B, SQ, DMODEL), F32),
        in_specs=[pl.BlockSpec(memory_space=pltpu.VMEM)] * 5,
        out_specs=pl.BlockSpec(memory_space=pltpu.VMEM),
        scratch_shapes=[
            pltpu.VMEM((DMODEL, QCOLS), BF16),
            pltpu.VMEM((QCOLS, DMODEL), BF16),
            pltpu.VMEM((B, SQ, QCOLS), BF16),
            pltpu.VMEM((N_DEV - 1, B, CH, DMODEL), BF16),
            pltpu.VMEM((N_DEV - 1, B, CH, DMODEL), BF16),
            pltpu.VMEM((B, CH, DMODEL), F32),
            pltpu.VMEM((B, CH, DMODEL), BF16),
            pltpu.VMEM((N_DEV - 1, B, CH, DMODEL), BF16),
            pltpu.SemaphoreType.DMA((N_DEV - 1,)),
            pltpu.SemaphoreType.DMA((N_DEV - 1,)),
            pltpu.SemaphoreType.DMA((N_DEV - 1,)),
            pltpu.SemaphoreType.DMA((N_DEV - 1,)),
        ],
        compiler_params=pltpu.CompilerParams(collective_id=0),
    )(x, wq_sl, K_ext, V_ext, wo_sl)
